# speedup vs baseline: 2.4007x; 2.4007x over previous
import functools

import jax
import jax.numpy as jnp
from jax.experimental import pallas as pl
from jax.experimental.pallas import tpu as pltpu
from jax.experimental.pallas import tpu_sc as plsc

_B = 16384
_EMB = 128
_WINDOW = 128
_BM = 1024


def _sc_gather(user_emb, item_emb, uid2d, iid2d):
    mesh = plsc.VectorSubcoreMesh(core_axis_name="c", subcore_axis_name="s")
    out_t = (jax.ShapeDtypeStruct((_B, _EMB), jnp.float32),
             jax.ShapeDtypeStruct((_B, _EMB), jnp.float32))

    @functools.partial(pl.kernel, out_type=out_t, mesh=mesh)
    def k(ue_hbm, ie_hbm, ui_hbm, ii_hbm, ou_hbm, oi_hbm):
        def body(ui_v, ii_v, ou_v, oi_v):
            pltpu.sync_copy(ue_hbm.at[ui_v.at[0]], ou_v)
            pltpu.sync_copy(ie_hbm.at[ii_v.at[0]], oi_v)

        pltpu.emit_pipeline(
            body,
            grid=(_B // _WINDOW,),
            in_specs=[
                pl.BlockSpec((1, _WINDOW), lambda i: (0, i)),
                pl.BlockSpec((1, _WINDOW), lambda i: (0, i)),
            ],
            out_specs=[
                pl.BlockSpec((_WINDOW, _EMB), lambda i: (i, 0)),
                pl.BlockSpec((_WINDOW, _EMB), lambda i: (i, 0)),
            ],
            core_axis_name=("c", "s"),
            dimension_semantics=(pltpu.PARALLEL,),
        )(ui_hbm, ii_hbm, ou_hbm, oi_hbm)

    return k(user_emb, item_emb, uid2d, iid2d)


def _mlp_body(ue, ie, w1u, w1i, b1, w2, b2, w3, b3, wo, bo, out):
    xu = ue[...].astype(jnp.bfloat16)
    xi = ie[...].astype(jnp.bfloat16)
    h = jnp.dot(xu, w1u[...], preferred_element_type=jnp.float32)
    h = h + jnp.dot(xi, w1i[...], preferred_element_type=jnp.float32)
    h = jnp.maximum(h + b1[...], 0.0).astype(jnp.bfloat16)
    h = jnp.dot(h, w2[...], preferred_element_type=jnp.float32)
    h = jnp.maximum(h + b2[...], 0.0).astype(jnp.bfloat16)
    h = jnp.dot(h, w3[...], preferred_element_type=jnp.float32)
    h = jnp.maximum(h + b3[...], 0.0).astype(jnp.bfloat16)
    lg = jnp.dot(h, wo[...], preferred_element_type=jnp.float32)
    out[...] = jax.nn.sigmoid(lg + bo[0, 0])


def _mlp(ue, ie, w1u, w1i, b1, w2, b2, w3, b3, wo, bo):
    def const(s):
        return pl.BlockSpec(s, lambda i: (0,) * len(s))

    return pl.pallas_call(
        _mlp_body,
        grid=(_B // _BM,),
        in_specs=[
            pl.BlockSpec((_BM, _EMB), lambda i: (i, 0)),
            pl.BlockSpec((_BM, _EMB), lambda i: (i, 0)),
            const((_EMB, 512)), const((_EMB, 512)), const((1, 512)),
            const((512, 256)), const((1, 256)),
            const((256, 128)), const((1, 128)),
            const((128, 1)), const((1, 1)),
        ],
        out_specs=pl.BlockSpec((_BM, 1), lambda i: (i, 0)),
        out_shape=jax.ShapeDtypeStruct((_B, 1), jnp.float32),
    )(ue, ie, w1u, w1i, b1, w2, b2, w3, b3, wo, bo)


def kernel(user_id, item_id, user_emb, item_emb, W1, b1, W2, b2, W3, b3,
           Wout, bout):
    ue, ie = _sc_gather(user_emb, item_emb,
                        user_id.reshape(1, _B), item_id.reshape(1, _B))
    bf = jnp.bfloat16
    return _mlp(ue, ie,
                W1[:_EMB].astype(bf), W1[_EMB:].astype(bf), b1.reshape(1, -1),
                W2.astype(bf), b2.reshape(1, -1),
                W3.astype(bf), b3.reshape(1, -1),
                Wout.astype(bf), bout.reshape(1, 1))
